# baseline (speedup 1.0000x reference)
import jax
import jax.numpy as jnp
from jax import lax
from jax.experimental import pallas as pl
from jax.experimental.pallas import tpu as pltpu
from jax.experimental.pallas import tpu_sc as plsc

NC = 2
NS = 16
NW = NC * NS
L = 16
D = 32
CH = 64
_PCOLS = 32768
NBUF = 3


def _pack_body(x_ref, y_ref):
    x = x_ref[...]
    for q in range(x.shape[1] // 512):
        off = 512 * q
        w = jnp.concatenate(
            [x[:, off + 128 * k:off + 128 * (k + 1)] for k in range(4)],
            axis=0)
        y_ref[pl.ds(128 * q, 128), :] = w.T


def _pack(table):
    n = table.shape[0]
    pcols = min(_PCOLS, ((n + 511) // 512) * 512)
    nb = (n + pcols - 1) // pcols
    return pl.pallas_call(
        _pack_body,
        grid=(nb,),
        in_specs=[pl.BlockSpec((32, pcols), lambda g: (0, g))],
        out_specs=pl.BlockSpec((pcols // 4, 128), lambda g: (g, 0)),
        out_shape=jax.ShapeDtypeStruct((nb * (pcols // 4), 128),
                                       jnp.float32),
    )(table.T)


def _prow(iv):
    return (lax.shift_left(lax.shift_right_logical(iv, 9), 7) + (iv & 127))


def _part_body(C,
               rel_p, tem_p,
               pos_r, pos_tem, neg_r, neg_tem,
               part_out,
               idx_s, row_s, cb_s, g0, g1, g2, rel_v, part_v,
               sem0, sem1, sem2, sem3):
    wid = lax.axis_index("s") * NC + lax.axis_index("c")
    base = wid * C
    nch = (2 * C) // CH
    cps = C // CH

    idx_in = (pos_tem, pos_r, neg_tem, neg_r)
    stg = [pltpu.async_copy(idx_in[t].at[pl.ds(base, C)], idx_s.at[t], sem3)
           for t in range(4)]
    relcp = pltpu.async_copy(rel_p, rel_v, sem3)
    for c in stg:
        c.wait()
    for t in range(4):
        def rows_step(v, _, t=t):
            sl = pl.ds(v * L, L)
            iv = idx_s[t, sl]
            cb_s[t, sl] = (lax.shift_right_logical(iv, 7) & 3) * D
            row_s[t, sl] = _prow(iv)
            return 0
        lax.fori_loop(0, C // L, rows_step, 0)

    sems = (sem0, sem1, sem2)
    bufs = (g0, g1, g2)

    def fire(c):
        side, cc = c // cps, c % cps
        return [pltpu.async_copy(
            tem_p.at[row_s.at[2 * side, pl.ds(cc * CH, CH)]],
            bufs[c % NBUF][0], sems[c % NBUF])]

    relcp.wait()
    inflight = [fire(c) for c in range(NBUF - 1)]

    for c in range(nch):
        inflight.append(fire(c + NBUF - 1) if c + NBUF - 1 < nch else [])
        for cp in inflight.pop(0):
            cp.wait()
        b = bufs[c % NBUF]
        side, cc = c // cps, c % cps
        soff = cc * CH

        def group(g, _):
            slot = g * L + lax.iota(jnp.int32, L)
            sl = pl.ds(soff + g * L, L)
            cm_cb = cb_s[2 * side, sl]
            rcb = cb_s[2 * side + 1, sl]
            rrow = row_s[2 * side + 1, sl]
            lane = lax.iota(jnp.int32, L)
            sbase = (c * CH + g * L + lane) * D

            def hex_step(o, _):
                j0 = o * 16
                for jj in range(16):
                    jc = (j0 + jj + lane) & (D - 1)
                    m = plsc.load_gather(b[0], [slot, cm_cb + jc])
                    r = plsc.load_gather(rel_v, [rrow, rcb + jc])
                    plsc.store_scatter(part_v, [sbase + jc], m + r)
                return 0

            lax.fori_loop(0, D // 16, hex_step, 0)
            return 0

        lax.fori_loop(0, CH // L, group, 0)

    pltpu.sync_copy(part_v.at[pl.ds(0, C * D)],
                    part_out.at[pl.ds(base * D, C * D)])
    pltpu.sync_copy(part_v.at[pl.ds(C * D, C * D)],
                    part_out.at[pl.ds((C * NW + base) * D, C * D)])


def _sc_body(C,
             ent_p, part,
             pos_h, pos_t, neg_h, neg_t,
             pos_out, neg_out,
             idx_s, row_s, cb_s, g0, g1, g2, out_v,
             sem0, sem1, sem2, sem3):
    wid = lax.axis_index("s") * NC + lax.axis_index("c")
    base = wid * C
    nch = (2 * C) // CH
    cps = C // CH

    idx_in = (pos_h, pos_t, neg_h, neg_t)
    stg = [pltpu.async_copy(idx_in[t].at[pl.ds(base, C)], idx_s.at[t], sem3)
           for t in range(4)]
    for c in stg:
        c.wait()
    for t in range(4):
        def rows_step(v, _, t=t):
            sl = pl.ds(v * L, L)
            iv = idx_s[t, sl]
            cb_s[t, sl] = (lax.shift_right_logical(iv, 7) & 3) * D
            row_s[t, sl] = _prow(iv)
            return 0
        lax.fori_loop(0, C // L, rows_step, 0)

    sems = (sem0, sem1, sem2)
    bufs = (g0, g1, g2)

    def fire(c):
        side, cc = c // cps, c % cps
        b = bufs[c % NBUF]
        poff = (side * C * NW + base + cc * CH) * D
        return [pltpu.async_copy(
            ent_p.at[row_s.at[2 * side + t, pl.ds(cc * CH, CH)]],
            b[t], sems[c % NBUF]) for t in range(2)] + [
            pltpu.async_copy(part.at[pl.ds(poff, CH * D)], b[2],
                             sems[c % NBUF])]

    inflight = [fire(c) for c in range(NBUF - 1)]

    for c in range(nch):
        inflight.append(fire(c + NBUF - 1) if c + NBUF - 1 < nch else [])
        for cp in inflight.pop(0):
            cp.wait()
        b = bufs[c % NBUF]
        side, cc = c // cps, c % cps
        ioff = side * 2
        soff = cc * CH

        def group(g, _):
            slot = g * L + lax.iota(jnp.int32, L)
            sl = pl.ds(soff + g * L, L)
            ch_cb = cb_s[ioff + 0, sl]
            ct_cb = cb_s[ioff + 1, sl]
            lane = lax.iota(jnp.int32, L)
            sbase = (g * L + lane) * D

            def hex_step(o, accs):
                a0, a1, a2, a3 = accs
                j0 = o * 16
                for jj in range(16):
                    jc = (j0 + jj + lane) & (D - 1)
                    h = plsc.load_gather(b[0], [slot, ch_cb + jc])
                    t_ = plsc.load_gather(b[1], [slot, ct_cb + jc])
                    p = plsc.load_gather(b[2], [sbase + jc])
                    v = jnp.abs(h + p - t_)
                    if jj % 4 == 0:
                        a0 = a0 + v
                    elif jj % 4 == 1:
                        a1 = a1 + v
                    elif jj % 4 == 2:
                        a2 = a2 + v
                    else:
                        a3 = a3 + v
                return (a0, a1, a2, a3)

            z = jnp.zeros((L,), jnp.float32)
            a0, a1, a2, a3 = lax.fori_loop(0, D // 16, hex_step,
                                           (z, z, z, z))
            out_v[pl.ds(c * CH + g * L, L)] = (a0 + a1) + (a2 + a3)
            return 0

        lax.fori_loop(0, CH // L, group, 0)

    pltpu.sync_copy(out_v.at[pl.ds(0, C)], pos_out.at[pl.ds(base, C)])
    pltpu.sync_copy(out_v.at[pl.ds(C, C)], neg_out.at[pl.ds(base, C)])


def kernel(pos_h, pos_t, pos_r, pos_tem, neg_h, neg_t, neg_r, neg_tem,
           ent_w, rel_w, tem_w):
    B = pos_h.shape[0]
    C = B // NW
    i32 = jnp.int32
    rel_p = _pack(rel_w)
    tem_p = _pack(tem_w)
    ent_p = _pack(ent_w)
    mesh = plsc.VectorSubcoreMesh(core_axis_name="c", subcore_axis_name="s")

    f1 = pl.kernel(
        lambda *refs: _part_body(C, *refs),
        out_type=jax.ShapeDtypeStruct((2 * B * D,), jnp.float32),
        mesh=mesh,
        scratch_types=[
            pltpu.VMEM((4, C), jnp.int32),
            pltpu.VMEM((4, C), jnp.int32),
            pltpu.VMEM((4, C), jnp.int32),
            (pltpu.VMEM((CH, 128), jnp.float32),),
            (pltpu.VMEM((CH, 128), jnp.float32),),
            (pltpu.VMEM((CH, 128), jnp.float32),),
            pltpu.VMEM((rel_p.shape[0], 128), jnp.float32),
            pltpu.VMEM((2 * C * D,), jnp.float32),
            pltpu.SemaphoreType.DMA,
            pltpu.SemaphoreType.DMA,
            pltpu.SemaphoreType.DMA,
            pltpu.SemaphoreType.DMA,
        ],
        compiler_params=pltpu.CompilerParams(needs_layout_passes=False),
    )
    part = f1(rel_p, tem_p,
              pos_r.astype(i32), pos_tem.astype(i32),
              neg_r.astype(i32), neg_tem.astype(i32))

    gset = lambda: (pltpu.VMEM((CH, 128), jnp.float32),
                    pltpu.VMEM((CH, 128), jnp.float32),
                    pltpu.VMEM((CH * D,), jnp.float32))
    f2 = pl.kernel(
        lambda *refs: _sc_body(C, *refs),
        out_type=(jax.ShapeDtypeStruct((B,), jnp.float32),
                  jax.ShapeDtypeStruct((B,), jnp.float32)),
        mesh=mesh,
        scratch_types=[
            pltpu.VMEM((4, C), jnp.int32),
            pltpu.VMEM((4, C), jnp.int32),
            pltpu.VMEM((4, C), jnp.int32),
            gset(),
            gset(),
            gset(),
            pltpu.VMEM((2 * C,), jnp.float32),
            pltpu.SemaphoreType.DMA,
            pltpu.SemaphoreType.DMA,
            pltpu.SemaphoreType.DMA,
            pltpu.SemaphoreType.DMA,
        ],
        compiler_params=pltpu.CompilerParams(needs_layout_passes=False),
    )
    return f2(ent_p, part,
              pos_h.astype(i32), pos_t.astype(i32),
              neg_h.astype(i32), neg_t.astype(i32))


# speedup vs baseline: 1.0300x; 1.0300x over previous
import jax
import jax.numpy as jnp
from jax import lax
from jax.experimental import pallas as pl
from jax.experimental.pallas import tpu as pltpu
from jax.experimental.pallas import tpu_sc as plsc

NC = 2
NS = 16
NW = NC * NS
L = 16
D = 32
CH = 64
_PCOLS = 65536
NBUF = 3


def _pack_body(x_ref, y_ref):
    x = x_ref[...]
    for q in range(x.shape[1] // 512):
        off = 512 * q
        w = jnp.concatenate(
            [x[:, off + 128 * k:off + 128 * (k + 1)] for k in range(4)],
            axis=0)
        y_ref[pl.ds(128 * q, 128), :] = w.T


def _pack(table):
    n = table.shape[0]
    pcols = min(_PCOLS, ((n + 511) // 512) * 512)
    nb = (n + pcols - 1) // pcols
    return pl.pallas_call(
        _pack_body,
        grid=(nb,),
        in_specs=[pl.BlockSpec((32, pcols), lambda g: (0, g))],
        out_specs=pl.BlockSpec((pcols // 4, 128), lambda g: (g, 0)),
        out_shape=jax.ShapeDtypeStruct((nb * (pcols // 4), 128),
                                       jnp.float32),
    )(table.T)


def _prow(iv):
    return (lax.shift_left(lax.shift_right_logical(iv, 9), 7) + (iv & 127))


def _part_body(C,
               rel_p, tem_p,
               pos_r, pos_tem, neg_r, neg_tem,
               part_out,
               idx_s, row_s, cb_s, g0, g1, g2, rel_v, part_v,
               sem0, sem1, sem2, sem3):
    wid = lax.axis_index("s") * NC + lax.axis_index("c")
    base = wid * C
    nch = (2 * C) // CH
    cps = C // CH

    idx_in = (pos_tem, pos_r, neg_tem, neg_r)
    stg = [pltpu.async_copy(idx_in[t].at[pl.ds(base, C)], idx_s.at[t], sem3)
           for t in range(4)]
    relcp = pltpu.async_copy(rel_p, rel_v, sem3)
    for c in stg:
        c.wait()
    for t in range(4):
        def rows_step(v, _, t=t):
            sl = pl.ds(v * L, L)
            iv = idx_s[t, sl]
            cb_s[t, sl] = (lax.shift_right_logical(iv, 7) & 3) * D
            row_s[t, sl] = _prow(iv)
            return 0
        lax.fori_loop(0, C // L, rows_step, 0)

    sems = (sem0, sem1, sem2)
    bufs = (g0, g1, g2)

    def fire(c):
        side, cc = c // cps, c % cps
        return [pltpu.async_copy(
            tem_p.at[row_s.at[2 * side, pl.ds(cc * CH, CH)]],
            bufs[c % NBUF][0], sems[c % NBUF])]

    relcp.wait()
    inflight = [fire(c) for c in range(NBUF - 1)]

    for c in range(nch):
        inflight.append(fire(c + NBUF - 1) if c + NBUF - 1 < nch else [])
        for cp in inflight.pop(0):
            cp.wait()
        b = bufs[c % NBUF]
        side, cc = c // cps, c % cps
        soff = cc * CH

        def group(g, _):
            slot = g * L + lax.iota(jnp.int32, L)
            sl = pl.ds(soff + g * L, L)
            cm_cb = cb_s[2 * side, sl]
            rcb = cb_s[2 * side + 1, sl]
            rrow = row_s[2 * side + 1, sl]
            lane = lax.iota(jnp.int32, L)
            sbase = (c * CH + g * L + lane) * D

            def hex_step(o, _):
                j0 = o * 16
                for jj in range(16):
                    jc = (j0 + jj + lane) & (D - 1)
                    m = plsc.load_gather(b[0], [slot, cm_cb + jc])
                    r = plsc.load_gather(rel_v, [rrow, rcb + jc])
                    plsc.store_scatter(part_v, [sbase + jc], m + r)
                return 0

            lax.fori_loop(0, D // 16, hex_step, 0)
            return 0

        lax.fori_loop(0, CH // L, group, 0)

    pltpu.sync_copy(part_v.at[pl.ds(0, C * D)],
                    part_out.at[pl.ds(base * D, C * D)])
    pltpu.sync_copy(part_v.at[pl.ds(C * D, C * D)],
                    part_out.at[pl.ds((C * NW + base) * D, C * D)])


def _sc_body(C,
             ent_p, part,
             pos_h, pos_t, neg_h, neg_t,
             pos_out, neg_out,
             idx_s, row_s, cb_s, g0, g1, g2, out_v,
             sem0, sem1, sem2, sem3):
    wid = lax.axis_index("s") * NC + lax.axis_index("c")
    base = wid * C
    nch = (2 * C) // CH
    cps = C // CH

    idx_in = (pos_h, pos_t, neg_h, neg_t)
    stg = [pltpu.async_copy(idx_in[t].at[pl.ds(base, C)], idx_s.at[t], sem3)
           for t in range(4)]
    for c in stg:
        c.wait()
    for t in range(4):
        def rows_step(v, _, t=t):
            sl = pl.ds(v * L, L)
            iv = idx_s[t, sl]
            cb_s[t, sl] = (lax.shift_right_logical(iv, 7) & 3) * D
            row_s[t, sl] = _prow(iv)
            return 0
        lax.fori_loop(0, C // L, rows_step, 0)

    sems = (sem0, sem1, sem2)
    bufs = (g0, g1, g2)

    def fire(c):
        side, cc = c // cps, c % cps
        b = bufs[c % NBUF]
        poff = (side * C * NW + base + cc * CH) * D
        return [pltpu.async_copy(
            ent_p.at[row_s.at[2 * side + t, pl.ds(cc * CH, CH)]],
            b[t], sems[c % NBUF]) for t in range(2)] + [
            pltpu.async_copy(part.at[pl.ds(poff, CH * D)], b[2],
                             sems[c % NBUF])]

    inflight = [fire(c) for c in range(NBUF - 1)]

    for c in range(nch):
        inflight.append(fire(c + NBUF - 1) if c + NBUF - 1 < nch else [])
        for cp in inflight.pop(0):
            cp.wait()
        b = bufs[c % NBUF]
        side, cc = c // cps, c % cps
        ioff = side * 2
        soff = cc * CH

        def group(g, _):
            slot = g * L + lax.iota(jnp.int32, L)
            sl = pl.ds(soff + g * L, L)
            ch_cb = cb_s[ioff + 0, sl]
            ct_cb = cb_s[ioff + 1, sl]
            lane = lax.iota(jnp.int32, L)
            sbase = (g * L + lane) * D

            def hex_step(o, accs):
                a0, a1, a2, a3 = accs
                j0 = o * 16
                for jj in range(16):
                    jc = (j0 + jj + lane) & (D - 1)
                    h = plsc.load_gather(b[0], [slot, ch_cb + jc])
                    t_ = plsc.load_gather(b[1], [slot, ct_cb + jc])
                    p = plsc.load_gather(b[2], [sbase + jc])
                    v = jnp.abs(h + p - t_)
                    if jj % 4 == 0:
                        a0 = a0 + v
                    elif jj % 4 == 1:
                        a1 = a1 + v
                    elif jj % 4 == 2:
                        a2 = a2 + v
                    else:
                        a3 = a3 + v
                return (a0, a1, a2, a3)

            z = jnp.zeros((L,), jnp.float32)
            a0, a1, a2, a3 = lax.fori_loop(0, D // 16, hex_step,
                                           (z, z, z, z))
            out_v[pl.ds(c * CH + g * L, L)] = (a0 + a1) + (a2 + a3)
            return 0

        lax.fori_loop(0, CH // L, group, 0)

    pltpu.sync_copy(out_v.at[pl.ds(0, C)], pos_out.at[pl.ds(base, C)])
    pltpu.sync_copy(out_v.at[pl.ds(C, C)], neg_out.at[pl.ds(base, C)])


def kernel(pos_h, pos_t, pos_r, pos_tem, neg_h, neg_t, neg_r, neg_tem,
           ent_w, rel_w, tem_w):
    B = pos_h.shape[0]
    C = B // NW
    i32 = jnp.int32
    rel_p = _pack(rel_w)
    tem_p = _pack(tem_w)
    ent_p = _pack(ent_w)
    mesh = plsc.VectorSubcoreMesh(core_axis_name="c", subcore_axis_name="s")

    f1 = pl.kernel(
        lambda *refs: _part_body(C, *refs),
        out_type=jax.ShapeDtypeStruct((2 * B * D,), jnp.float32),
        mesh=mesh,
        scratch_types=[
            pltpu.VMEM((4, C), jnp.int32),
            pltpu.VMEM((4, C), jnp.int32),
            pltpu.VMEM((4, C), jnp.int32),
            (pltpu.VMEM((CH, 128), jnp.float32),),
            (pltpu.VMEM((CH, 128), jnp.float32),),
            (pltpu.VMEM((CH, 128), jnp.float32),),
            pltpu.VMEM((rel_p.shape[0], 128), jnp.float32),
            pltpu.VMEM((2 * C * D,), jnp.float32),
            pltpu.SemaphoreType.DMA,
            pltpu.SemaphoreType.DMA,
            pltpu.SemaphoreType.DMA,
            pltpu.SemaphoreType.DMA,
        ],
        compiler_params=pltpu.CompilerParams(needs_layout_passes=False),
    )
    part = f1(rel_p, tem_p,
              pos_r.astype(i32), pos_tem.astype(i32),
              neg_r.astype(i32), neg_tem.astype(i32))

    gset = lambda: (pltpu.VMEM((CH, 128), jnp.float32),
                    pltpu.VMEM((CH, 128), jnp.float32),
                    pltpu.VMEM((CH * D,), jnp.float32))
    f2 = pl.kernel(
        lambda *refs: _sc_body(C, *refs),
        out_type=(jax.ShapeDtypeStruct((B,), jnp.float32),
                  jax.ShapeDtypeStruct((B,), jnp.float32)),
        mesh=mesh,
        scratch_types=[
            pltpu.VMEM((4, C), jnp.int32),
            pltpu.VMEM((4, C), jnp.int32),
            pltpu.VMEM((4, C), jnp.int32),
            gset(),
            gset(),
            gset(),
            pltpu.VMEM((2 * C,), jnp.float32),
            pltpu.SemaphoreType.DMA,
            pltpu.SemaphoreType.DMA,
            pltpu.SemaphoreType.DMA,
            pltpu.SemaphoreType.DMA,
        ],
        compiler_params=pltpu.CompilerParams(needs_layout_passes=False),
    )
    return f2(ent_p, part,
              pos_h.astype(i32), pos_t.astype(i32),
              neg_h.astype(i32), neg_t.astype(i32))


# speedup vs baseline: 1.0338x; 1.0037x over previous
import jax
import jax.numpy as jnp
from jax import lax
from jax.experimental import pallas as pl
from jax.experimental.pallas import tpu as pltpu
from jax.experimental.pallas import tpu_sc as plsc

NC = 2
NS = 16
NW = NC * NS
L = 16
D = 32
CH = 64
CH2 = 128
_PCOLS = 65536
NBUF = 3


def _pack_body(x_ref, y_ref):
    x = x_ref[...]
    for q in range(x.shape[1] // 512):
        off = 512 * q
        w = jnp.concatenate(
            [x[:, off + 128 * k:off + 128 * (k + 1)] for k in range(4)],
            axis=0)
        y_ref[pl.ds(128 * q, 128), :] = w.T


def _pack(table):
    n = table.shape[0]
    pcols = min(_PCOLS, ((n + 511) // 512) * 512)
    nb = (n + pcols - 1) // pcols
    return pl.pallas_call(
        _pack_body,
        grid=(nb,),
        in_specs=[pl.BlockSpec((32, pcols), lambda g: (0, g))],
        out_specs=pl.BlockSpec((pcols // 4, 128), lambda g: (g, 0)),
        out_shape=jax.ShapeDtypeStruct((nb * (pcols // 4), 128),
                                       jnp.float32),
    )(table.T)


def _prow(iv):
    return (lax.shift_left(lax.shift_right_logical(iv, 9), 7) + (iv & 127))


def _part_body(C,
               rel_p, tem_p,
               pos_r, pos_tem, neg_r, neg_tem,
               part_out,
               idx_s, row_s, cb_s, g0, g1, g2, rel_v, part_v,
               sem0, sem1, sem2, sem3):
    wid = lax.axis_index("s") * NC + lax.axis_index("c")
    base = wid * C
    nch = (2 * C) // CH
    cps = C // CH

    idx_in = (pos_tem, pos_r, neg_tem, neg_r)
    stg = [pltpu.async_copy(idx_in[t].at[pl.ds(base, C)], idx_s.at[t], sem3)
           for t in range(4)]
    relcp = pltpu.async_copy(rel_p, rel_v, sem3)
    for c in stg:
        c.wait()
    for t in range(4):
        def rows_step(v, _, t=t):
            sl = pl.ds(v * L, L)
            iv = idx_s[t, sl]
            cb_s[t, sl] = (lax.shift_right_logical(iv, 7) & 3) * D
            row_s[t, sl] = _prow(iv)
            return 0
        lax.fori_loop(0, C // L, rows_step, 0)

    sems = (sem0, sem1, sem2)
    bufs = (g0, g1, g2)

    def fire(c):
        side, cc = c // cps, c % cps
        return [pltpu.async_copy(
            tem_p.at[row_s.at[2 * side, pl.ds(cc * CH, CH)]],
            bufs[c % NBUF][0], sems[c % NBUF])]

    relcp.wait()
    inflight = [fire(c) for c in range(NBUF - 1)]

    for c in range(nch):
        inflight.append(fire(c + NBUF - 1) if c + NBUF - 1 < nch else [])
        for cp in inflight.pop(0):
            cp.wait()
        b = bufs[c % NBUF]
        side, cc = c // cps, c % cps
        soff = cc * CH

        def group(g, _):
            slot = g * L + lax.iota(jnp.int32, L)
            sl = pl.ds(soff + g * L, L)
            cm_cb = cb_s[2 * side, sl]
            rcb = cb_s[2 * side + 1, sl]
            rrow = row_s[2 * side + 1, sl]
            lane = lax.iota(jnp.int32, L)
            sbase = (c * CH + g * L + lane) * D

            def hex_step(o, _):
                j0 = o * 16
                for jj in range(16):
                    jc = (j0 + jj + lane) & (D - 1)
                    m = plsc.load_gather(b[0], [slot, cm_cb + jc])
                    r = plsc.load_gather(rel_v, [rrow, rcb + jc])
                    plsc.store_scatter(part_v, [sbase + jc], m + r)
                return 0

            lax.fori_loop(0, D // 16, hex_step, 0)
            return 0

        lax.fori_loop(0, CH // L, group, 0)

    pltpu.sync_copy(part_v.at[pl.ds(0, C * D)],
                    part_out.at[pl.ds(base * D, C * D)])
    pltpu.sync_copy(part_v.at[pl.ds(C * D, C * D)],
                    part_out.at[pl.ds((C * NW + base) * D, C * D)])


def _sc_body(C,
             ent_p, part,
             pos_h, pos_t, neg_h, neg_t,
             pos_out, neg_out,
             idx_s, row_s, cb_s, g0, g1, g2, out_v,
             sem0, sem1, sem2, sem3):
    wid = lax.axis_index("s") * NC + lax.axis_index("c")
    base = wid * C
    nch = (2 * C) // CH2
    cps = C // CH2

    idx_in = (pos_h, pos_t, neg_h, neg_t)
    stg = [pltpu.async_copy(idx_in[t].at[pl.ds(base, C)], idx_s.at[t], sem3)
           for t in range(4)]
    for c in stg:
        c.wait()
    for t in range(4):
        def rows_step(v, _, t=t):
            sl = pl.ds(v * L, L)
            iv = idx_s[t, sl]
            cb_s[t, sl] = (lax.shift_right_logical(iv, 7) & 3) * D
            row_s[t, sl] = _prow(iv)
            return 0
        lax.fori_loop(0, C // L, rows_step, 0)

    sems = (sem0, sem1, sem2)
    bufs = (g0, g1, g2)

    def fire(c):
        side, cc = c // cps, c % cps
        b = bufs[c % NBUF]
        poff = (side * C * NW + base + cc * CH2) * D
        return [pltpu.async_copy(
            ent_p.at[row_s.at[2 * side + t, pl.ds(cc * CH2, CH2)]],
            b[t], sems[c % NBUF]) for t in range(2)] + [
            pltpu.async_copy(part.at[pl.ds(poff, CH2 * D)], b[2],
                             sems[c % NBUF])]

    inflight = [fire(c) for c in range(NBUF - 1)]

    for c in range(nch):
        inflight.append(fire(c + NBUF - 1) if c + NBUF - 1 < nch else [])
        for cp in inflight.pop(0):
            cp.wait()
        b = bufs[c % NBUF]
        side, cc = c // cps, c % cps
        ioff = side * 2
        soff = cc * CH2

        def group(g, _):
            slot = g * L + lax.iota(jnp.int32, L)
            sl = pl.ds(soff + g * L, L)
            ch_cb = cb_s[ioff + 0, sl]
            ct_cb = cb_s[ioff + 1, sl]
            lane = lax.iota(jnp.int32, L)
            sbase = (g * L + lane) * D

            def hex_step(o, accs):
                a0, a1, a2, a3 = accs
                j0 = o * 16
                for jj in range(16):
                    jc = (j0 + jj + lane) & (D - 1)
                    h = plsc.load_gather(b[0], [slot, ch_cb + jc])
                    t_ = plsc.load_gather(b[1], [slot, ct_cb + jc])
                    p = plsc.load_gather(b[2], [sbase + jc])
                    v = jnp.abs(h + p - t_)
                    if jj % 4 == 0:
                        a0 = a0 + v
                    elif jj % 4 == 1:
                        a1 = a1 + v
                    elif jj % 4 == 2:
                        a2 = a2 + v
                    else:
                        a3 = a3 + v
                return (a0, a1, a2, a3)

            z = jnp.zeros((L,), jnp.float32)
            a0, a1, a2, a3 = lax.fori_loop(0, D // 16, hex_step,
                                           (z, z, z, z))
            out_v[pl.ds(c * CH2 + g * L, L)] = (a0 + a1) + (a2 + a3)
            return 0

        lax.fori_loop(0, CH2 // L, group, 0)

    pltpu.sync_copy(out_v.at[pl.ds(0, C)], pos_out.at[pl.ds(base, C)])
    pltpu.sync_copy(out_v.at[pl.ds(C, C)], neg_out.at[pl.ds(base, C)])


def kernel(pos_h, pos_t, pos_r, pos_tem, neg_h, neg_t, neg_r, neg_tem,
           ent_w, rel_w, tem_w):
    B = pos_h.shape[0]
    C = B // NW
    i32 = jnp.int32
    rel_p = _pack(rel_w)
    tem_p = _pack(tem_w)
    ent_p = _pack(ent_w)
    mesh = plsc.VectorSubcoreMesh(core_axis_name="c", subcore_axis_name="s")

    f1 = pl.kernel(
        lambda *refs: _part_body(C, *refs),
        out_type=jax.ShapeDtypeStruct((2 * B * D,), jnp.float32),
        mesh=mesh,
        scratch_types=[
            pltpu.VMEM((4, C), jnp.int32),
            pltpu.VMEM((4, C), jnp.int32),
            pltpu.VMEM((4, C), jnp.int32),
            (pltpu.VMEM((CH, 128), jnp.float32),),
            (pltpu.VMEM((CH, 128), jnp.float32),),
            (pltpu.VMEM((CH, 128), jnp.float32),),
            pltpu.VMEM((rel_p.shape[0], 128), jnp.float32),
            pltpu.VMEM((2 * C * D,), jnp.float32),
            pltpu.SemaphoreType.DMA,
            pltpu.SemaphoreType.DMA,
            pltpu.SemaphoreType.DMA,
            pltpu.SemaphoreType.DMA,
        ],
        compiler_params=pltpu.CompilerParams(needs_layout_passes=False),
    )
    part = f1(rel_p, tem_p,
              pos_r.astype(i32), pos_tem.astype(i32),
              neg_r.astype(i32), neg_tem.astype(i32))

    gset = lambda: (pltpu.VMEM((CH2, 128), jnp.float32),
                    pltpu.VMEM((CH2, 128), jnp.float32),
                    pltpu.VMEM((CH2 * D,), jnp.float32))
    f2 = pl.kernel(
        lambda *refs: _sc_body(C, *refs),
        out_type=(jax.ShapeDtypeStruct((B,), jnp.float32),
                  jax.ShapeDtypeStruct((B,), jnp.float32)),
        mesh=mesh,
        scratch_types=[
            pltpu.VMEM((4, C), jnp.int32),
            pltpu.VMEM((4, C), jnp.int32),
            pltpu.VMEM((4, C), jnp.int32),
            gset(),
            gset(),
            gset(),
            pltpu.VMEM((2 * C,), jnp.float32),
            pltpu.SemaphoreType.DMA,
            pltpu.SemaphoreType.DMA,
            pltpu.SemaphoreType.DMA,
            pltpu.SemaphoreType.DMA,
        ],
        compiler_params=pltpu.CompilerParams(needs_layout_passes=False),
    )
    return f2(ent_p, part,
              pos_h.astype(i32), pos_t.astype(i32),
              neg_h.astype(i32), neg_t.astype(i32))
